# baseline (speedup 1.0000x reference)
import functools

import jax
import jax.numpy as jnp
from jax import lax
from jax.experimental import pallas as pl
from jax.experimental.pallas import tpu as pltpu
from jax.experimental.pallas import tpu_sc as plsc

_E = 8
_NEG = -3.0e38
_NW = 32
_L = 16


def _proj_body(x_ref, w_ref, b_ref, o_ref):
    xb = x_ref[...]
    m = jnp.dot(xb, w_ref[...], preferred_element_type=jnp.float32)
    m = m + b_ref[...]
    o_ref[...] = m.T


def _tc_project(x2, W, b, N, D):
    BT = 4096
    return pl.pallas_call(
        _proj_body,
        grid=(N // BT,),
        in_specs=[
            pl.BlockSpec((BT, D), lambda i: (i, 0)),
            pl.BlockSpec((D, 2 * _E), lambda i: (0, 0)),
            pl.BlockSpec((1, 2 * _E), lambda i: (0, 0)),
        ],
        out_specs=pl.BlockSpec((2 * _E, BT), lambda i: (0, i)),
        out_shape=jax.ShapeDtypeStruct((2 * _E, N), jnp.float32),
    )(x2, W, b)


def _make_sc_route(N):
    chunk = N // _NW
    mesh = plsc.VectorSubcoreMesh(core_axis_name="c", subcore_axis_name="s")

    @functools.partial(
        pl.kernel,
        mesh=mesh,
        out_type=jax.ShapeDtypeStruct((N,), jnp.float32),
        scratch_types=[
            pltpu.VMEM((2 * _E, chunk), jnp.float32),
            pltpu.VMEM((chunk,), jnp.float32),
        ],
    )
    def _route(mt_hbm, out_hbm, buf, obuf):
        wid = lax.axis_index("s") * 2 + lax.axis_index("c")
        base = wid * chunk
        pltpu.sync_copy(mt_hbm.at[:, pl.ds(base, chunk)], buf)

        def body(g, carry):
            sl = pl.ds(g * _L, _L)
            l = [buf[e, sl] for e in range(_E)]
            z = [buf[_E + e, sl] for e in range(_E)]
            one = jnp.ones((_L,), jnp.float32)
            zero = jnp.zeros((_L,), jnp.float32)
            m1 = l[0]
            for e in range(1, _E):
                m1 = jnp.maximum(m1, l[e])
            eq = [jnp.where(l[e] == m1, one, zero) for e in range(_E)]
            seen = eq[0]
            fo = [eq[0]]
            for e in range(1, _E):
                fo.append(eq[e] * (one - seen))
                seen = jnp.maximum(seen, eq[e])
            l2 = [l[e] + fo[e] * _NEG for e in range(_E)]
            m2 = l2[0]
            for e in range(1, _E):
                m2 = jnp.maximum(m2, l2[e])
            eq2 = [jnp.where(l2[e] == m2, one, zero) for e in range(_E)]
            seen2 = eq2[0]
            fo2 = [eq2[0]]
            for e in range(1, _E):
                fo2.append(eq2[e] * (one - seen2))
                seen2 = jnp.maximum(seen2, eq2[e])
            num = zero
            den = zero
            for e in range(_E):
                sel = fo[e] + fo2[e]
                ex = sel * jnp.exp(l[e] - m1)
                sig = 1.0 / (1.0 + jnp.exp(-z[e]))
                num = num + ex * sig
                den = den + ex
            obuf[sl] = num / den
            return carry

        lax.fori_loop(0, chunk // _L, body, 0)
        pltpu.sync_copy(obuf, out_hbm.at[pl.ds(base, chunk)])

    return _route


@jax.jit
def kernel(x, Wc, bc, We, be):
    B, S, D = x.shape
    N = B * S
    x2 = x.reshape(N, D)
    W = jnp.concatenate([Wc, We.T], axis=1)
    b = jnp.concatenate([bc, be]).reshape(1, 2 * _E)

    C = 2
    NC = N // C
    route = _make_sc_route(NC)
    outs = []
    for c in range(C):
        mt_c = _tc_project(x2[c * NC:(c + 1) * NC], W, b, NC, D)
        outs.append(route(mt_c))
    out = jnp.concatenate(outs)
    return out.reshape(B, S, 1)


# speedup vs baseline: 3.5091x; 3.5091x over previous
import functools

import jax
import jax.numpy as jnp
from jax.experimental import pallas as pl

_E = 8
_NEG = -3.0e38


def _fused_body(x_ref, w_ref, b_ref, o_ref):
    xb = x_ref[...]
    m = jnp.dot(xb, w_ref[...], preferred_element_type=jnp.float32)
    m = m + b_ref[...]
    mt = m.T
    logits = mt[:_E, :]
    sig = jax.nn.sigmoid(mt[_E:, :])

    iota = jax.lax.broadcasted_iota(jnp.int32, logits.shape, 0)
    m1 = jnp.max(logits, axis=0, keepdims=True)
    eq1 = logits == m1
    i1 = jnp.min(jnp.where(eq1, iota, _E), axis=0, keepdims=True)
    sel1 = iota == i1
    masked = jnp.where(sel1, _NEG, logits)
    m2 = jnp.max(masked, axis=0, keepdims=True)
    eq2 = masked == m2
    i2 = jnp.min(jnp.where(eq2, iota, _E), axis=0, keepdims=True)
    sel = sel1 | (iota == i2)

    w = jnp.where(sel, jnp.exp(logits - m1), 0.0)
    denom = jnp.sum(w, axis=0, keepdims=True)
    o_ref[...] = jnp.sum(w * sig, axis=0, keepdims=True) / denom


@functools.partial(jax.jit, static_argnames=("interpret",))
def kernel(x, Wc, bc, We, be, interpret=False):
    B, S, D = x.shape
    N = B * S
    x2 = x.reshape(N, D)
    W = jnp.concatenate([Wc, We.T], axis=1)
    b = jnp.concatenate([bc, be]).reshape(1, 2 * _E)

    BT = 4096
    out = pl.pallas_call(
        _fused_body,
        grid=(N // BT,),
        in_specs=[
            pl.BlockSpec((BT, D), lambda i: (i, 0)),
            pl.BlockSpec((D, 2 * _E), lambda i: (0, 0)),
            pl.BlockSpec((1, 2 * _E), lambda i: (0, 0)),
        ],
        out_specs=pl.BlockSpec((1, BT), lambda i: (0, i)),
        out_shape=jax.ShapeDtypeStruct((1, N), jnp.float32),
        interpret=interpret,
    )(x2, W, b)
    return out.reshape(B, S, 1)
